# baseline (speedup 1.0000x reference)
import jax
import jax.numpy as jnp
import numpy as np
from jax.experimental import pallas as pl
from jax.experimental.pallas import tpu as pltpu

_I1 = np.arange(2).reshape(2, 1)
_I2 = np.array([[2, 3, 4], [0, 6, 1]], dtype=np.int32)


def _body(g_ref, x_ref, out_ref):
    out_ref[...] = g_ref[...]


def kernel(x):
    g = x[jnp.asarray(_I1, jnp.int32), jnp.asarray(_I2, jnp.int32)]
    return pl.pallas_call(
        _body,
        in_specs=[
            pl.BlockSpec(memory_space=pltpu.VMEM),
            pl.BlockSpec(memory_space=pl.ANY),
        ],
        out_specs=pl.BlockSpec(memory_space=pltpu.VMEM),
        out_shape=jax.ShapeDtypeStruct((2, 3, 128), jnp.float32),
    )(g, x)


# speedup vs baseline: 8.3762x; 8.3762x over previous
import jax
import jax.numpy as jnp
from jax.experimental import pallas as pl


def _gather_body(x_ref, out_ref):
    out_ref[0, :, :] = x_ref[0, 2:5, :]
    out_ref[1, 0:1, :] = x_ref[1, 0:1, :]
    out_ref[1, 1:2, :] = x_ref[1, 6:7, :]
    out_ref[1, 2:3, :] = x_ref[1, 1:2, :]


def kernel(x):
    slab = jax.lax.slice(x, (0, 0, 0), (2, 12, 128))
    return pl.pallas_call(
        _gather_body,
        out_shape=jax.ShapeDtypeStruct((2, 3, 128), jnp.float32),
    )(slab)


# speedup vs baseline: 8.5502x; 1.0208x over previous
import jax
import jax.numpy as jnp
from jax.experimental import pallas as pl


def _gather_body(x_ref, out_ref):
    out_ref[0, :, :] = x_ref[0, 2:5, :]
    out_ref[1, 0:1, :] = x_ref[1, 0:1, :]
    out_ref[1, 1:2, :] = x_ref[1, 6:7, :]
    out_ref[1, 2:3, :] = x_ref[1, 1:2, :]


def kernel(x):
    slab = jax.lax.slice(x, (0, 0, 0), (2, 8, 128))
    return pl.pallas_call(
        _gather_body,
        out_shape=jax.ShapeDtypeStruct((2, 3, 128), jnp.float32),
    )(slab)
